# Initial kernel scaffold; baseline (speedup 1.0000x reference)
import jax
import jax.numpy as jnp
from jax.experimental import pallas as pl


def kernel(x_note, edge_index_onset, edge_index_voice, params):
    raise NotImplementedError("write your pallas kernel here")


# speedup vs baseline: 3.2635x; 3.2635x over previous
import functools

import jax
import jax.numpy as jnp
from jax import lax
from jax.experimental import pallas as pl
from jax.experimental.pallas import tpu as pltpu
from jax.experimental.pallas import tpu_sc as plsc

N = 10000
D = 128
HID = 256
CLF_H = 64
OUT = 3
EPS = 1e-5
F32 = jnp.float32

NS = 16
CH = 80
DP = D + 16
ROWS_PT = 624
TAIL = N - NS * ROWS_PT


def _zero_vmem(ref, nrows, width):
    z = jnp.zeros((16,), F32)

    def body(i, _):
        for k in range(width // 16):
            ref[i, pl.ds(k * 16, 16)] = z
        return 0

    lax.fori_loop(0, nrows, body, 0)


def _fill_vmem(ref, nrows, width, val):
    v = jnp.full((16,), val, F32)

    def body(i, _):
        for k in range(width // 16):
            ref[i, pl.ds(k * 16, 16)] = v
        return 0

    lax.fori_loop(0, nrows, body, 0)


def _copy_rows(src, dst, dst_base, nrows, chunk):
    full, rem = divmod(nrows, chunk)
    for t in range(full):
        pltpu.sync_copy(src, dst.at[pl.ds(dst_base + t * chunk, chunk), :])
    if rem:
        pltpu.sync_copy(src.at[pl.ds(0, rem), :],
                        dst.at[pl.ds(dst_base + full * chunk, rem), :])


def _zero_own_rows(acc, zsrc, s):
    _copy_rows(zsrc, acc, s * ROWS_PT, ROWS_PT, CH)
    pl.when(s == NS - 1)(lambda: pltpu.sync_copy(
        zsrc.at[pl.ds(0, TAIL), :], acc.at[pl.ds(N - TAIL, TAIL), :]))


def _dump_own_rows(acc, out, s):
    base = s * ROWS_PT
    pltpu.sync_copy(acc.at[pl.ds(base, ROWS_PT), :],
                    out.at[pl.ds(base, ROWS_PT), :])
    pl.when(s == NS - 1)(lambda: pltpu.sync_copy(
        acc.at[pl.ds(N - TAIL, TAIL), :], out.at[pl.ds(N - TAIL, TAIL), :]))


def _seg_accumulate(tab, src, dst, acc, src_v, dst_v, rows_v, sem, s,
                    edge_base, per_tile, accc=None, ones_v=None):
    nch = per_tile // CH

    def body(j, _):
        base = edge_base + s * per_tile + j * CH
        pltpu.sync_copy(src.at[pl.ds(base, CH)], src_v)
        pltpu.sync_copy(dst.at[pl.ds(base, CH)], dst_v)
        pltpu.async_copy(tab.at[src_v], rows_v, sem).wait()
        pltpu.sync_copy(rows_v, acc.at[dst_v], add=True)
        if accc is not None:
            pltpu.sync_copy(ones_v, accc.at[dst_v], add=True)
        return 0

    lax.fori_loop(0, nch, body, 0)


def _sc_l0_body(x_hbm, src_cat, dst_cat, s_on_o, s_vo_o, c_on_o, c_vo_o,
                acc, src_v, dst_v, rows_v, sem):
    c = lax.axis_index("c")
    s = lax.axis_index("s")
    e_total = src_cat.shape[0] // 2
    per_tile = e_total // NS
    nch = per_tile // CH

    _zero_vmem(rows_v, CH, D)
    _zero_own_rows(acc, rows_v, s)
    plsc.subcore_barrier()

    _seg_accumulate(x_hbm, src_cat, dst_cat, acc, src_v, dst_v, rows_v, sem, s,
                    c * e_total, per_tile)
    plsc.subcore_barrier()

    pl.when(c == 0)(lambda: _dump_own_rows(acc, s_on_o, s))
    pl.when(c == 1)(lambda: _dump_own_rows(acc, s_vo_o, s))

    _zero_vmem(rows_v, CH, D)
    _zero_own_rows(acc, rows_v, s)
    _fill_vmem(rows_v, CH, D, 1.0)
    plsc.subcore_barrier()

    def cbody(j, _):
        base = c * e_total + s * per_tile + j * CH
        pltpu.sync_copy(dst_cat.at[pl.ds(base, CH)], dst_v)
        pltpu.sync_copy(rows_v, acc.at[dst_v], add=True)
        return 0

    lax.fori_loop(0, nch, cbody, 0)
    plsc.subcore_barrier()
    pl.when(c == 0)(lambda: _dump_own_rows(acc, c_on_o, s))
    pl.when(c == 1)(lambda: _dump_own_rows(acc, c_vo_o, s))


def _sc_l1_body(h0_hbm, h1_hbm, src_on, dst_on, src_vo, dst_vo,
                on0_o, on1_o, vo0_o, vo1_o,
                acc, src_v, dst_v, rows_v, sem):
    c = lax.axis_index("c")
    s = lax.axis_index("s")
    e_total = src_on.shape[0]
    per_tile = e_total // NS

    for task, (src, dst, out0, out1) in enumerate(
            ((src_on, dst_on, on0_o, on1_o), (src_vo, dst_vo, vo0_o, vo1_o))):
        _zero_vmem(rows_v, CH, D)
        _zero_own_rows(acc, rows_v, s)
        plsc.subcore_barrier()

        def run(tab, src=src, dst=dst):
            _seg_accumulate(tab, src, dst, acc, src_v, dst_v, rows_v, sem, s,
                            0, per_tile)

        pl.when(c == 0)(lambda: run(h0_hbm))
        pl.when(c == 1)(lambda: run(h1_hbm))
        plsc.subcore_barrier()
        pl.when(c == 0)(lambda out0=out0: _dump_own_rows(acc, out0, s))
        pl.when(c == 1)(lambda out1=out1: _dump_own_rows(acc, out1, s))


def _sc_pool_body(h_hbm, src_on, dst_on, p0_o, p1_o,
                  acc, src_v, dst_v, rows_v, sem):
    c = lax.axis_index("c")
    s = lax.axis_index("s")
    e_total = src_on.shape[0]
    e_half = e_total // 2
    per_tile = e_half // NS

    _zero_vmem(rows_v, CH, D)
    _zero_own_rows(acc, rows_v, s)
    plsc.subcore_barrier()
    _seg_accumulate(h_hbm, src_on, dst_on, acc, src_v, dst_v, rows_v, sem, s,
                    c * e_half, per_tile)
    plsc.subcore_barrier()
    pl.when(c == 0)(lambda: _dump_own_rows(acc, p0_o, s))
    pl.when(c == 1)(lambda: _dump_own_rows(acc, p1_o, s))


def _make_sc_kernels():
    mesh = plsc.VectorSubcoreMesh(core_axis_name="c", subcore_axis_name="s",
                                  num_cores=2, num_subcores=NS)
    f = jax.ShapeDtypeStruct
    nd = f((N, D), F32)
    nc = f((N, 16), F32)

    del nc
    l0 = pl.kernel(
        _sc_l0_body, out_type=(nd, nd, nd, nd), mesh=mesh,
        scratch_types=[
            pltpu.VMEM_SHARED((N, D), F32),
            pltpu.VMEM((CH,), jnp.int32), pltpu.VMEM((CH,), jnp.int32),
            pltpu.VMEM((CH, D), F32),
            pltpu.SemaphoreType.DMA,
        ])
    l1 = pl.kernel(
        _sc_l1_body, out_type=(nd, nd, nd, nd), mesh=mesh,
        scratch_types=[
            pltpu.VMEM_SHARED((N, D), F32),
            pltpu.VMEM((CH,), jnp.int32), pltpu.VMEM((CH,), jnp.int32),
            pltpu.VMEM((CH, D), F32),
            pltpu.SemaphoreType.DMA,
        ])
    pool = pl.kernel(
        _sc_pool_body, out_type=(nd, nd), mesh=mesh,
        scratch_types=[
            pltpu.VMEM_SHARED((N, D), F32),
            pltpu.VMEM((CH,), jnp.int32), pltpu.VMEM((CH,), jnp.int32),
            pltpu.VMEM((CH, D), F32),
            pltpu.SemaphoreType.DMA,
        ])
    return l0, l1, pool


_SC_L0, _SC_L1, _SC_POOL = _make_sc_kernels()


BR = 1000


def _tc_a_body(s_on, c_on, s_vo, c_vo, x,
               wn_on, wr_on, wn_vo, wr_vo, b, h0_o, h1_o):
    agg_on = s_on[:] / jnp.maximum(c_on[:, :1], 1.0)
    agg_vo = s_vo[:] / jnp.maximum(c_vo[:, :1], 1.0)
    h = (jnp.dot(agg_on, wn_on[:], preferred_element_type=F32)
         + jnp.dot(agg_vo, wn_vo[:], preferred_element_type=F32)
         + jnp.dot(x[:], wr_on[:] + wr_vo[:], preferred_element_type=F32)
         + b[:])
    h = jnp.maximum(h, 0.0)
    h0_o[:] = h[:, :D]
    h1_o[:] = h[:, D:]


def _tc_b_body(on0, on1, vo0, vo1, c_on, c_vo, h0, h1,
               wn_on, wr_on, wn_vo, wr_vo, b, lin_w, lin_b, h3_o):
    r_on = 1.0 / jnp.maximum(c_on[:, :1], 1.0)
    r_vo = 1.0 / jnp.maximum(c_vo[:, :1], 1.0)
    agg_on = jnp.concatenate([on0[:] * r_on, on1[:] * r_on], axis=-1)
    agg_vo = jnp.concatenate([vo0[:] * r_vo, vo1[:] * r_vo], axis=-1)
    h = jnp.concatenate([h0[:], h1[:]], axis=-1)
    z = (jnp.dot(agg_on, wn_on[:], preferred_element_type=F32)
         + jnp.dot(agg_vo, wn_vo[:], preferred_element_type=F32)
         + jnp.dot(h, wr_on[:] + wr_vo[:], preferred_element_type=F32)
         + b[:])
    z = jnp.maximum(z, 0.0)
    h3_o[:] = jnp.dot(z, lin_w[:], preferred_element_type=F32) + lin_b[:]


def _ln(x, g, b):
    m = jnp.mean(x, axis=-1, keepdims=True)
    v = jnp.mean((x - m) ** 2, axis=-1, keepdims=True)
    return (x - m) / jnp.sqrt(v + EPS) * g + b


def _tc_c_body(p0, p1, c_on, h3, norm_g, norm_b, pm_w1, pm_b1, pm_g, pm_b,
               pm_w2, pm_b2, cw1, cb1, bn_g, bn_b, bn_rm, bn_rv, cw2, cb2,
               out_o):
    pooled = (p0[:] + p1[:] + h3[:]) / jnp.maximum(c_on[:, :1], 1.0)
    h = _ln(pooled, norm_g[:], norm_b[:])
    z = jnp.maximum(jnp.dot(h, pm_w1[:], preferred_element_type=F32) + pm_b1[:], 0.0)
    z = _ln(z, pm_g[:], pm_b[:])
    z = jnp.dot(z, pm_w2[:], preferred_element_type=F32) + pm_b2[:]
    c = jnp.maximum(jnp.dot(z, cw1[:], preferred_element_type=F32) + cb1[:], 0.0)
    c = (c - bn_rm[:]) / jnp.sqrt(bn_rv[:] + EPS) * bn_g[:] + bn_b[:]
    logits = jnp.dot(c, cw2[:], preferred_element_type=F32) + cb2[:]
    m = jnp.max(logits, axis=-1, keepdims=True)
    e = jnp.exp(logits - m)
    out_o[:] = e / jnp.sum(e, axis=-1, keepdims=True)


def _row_spec(w):
    return pl.BlockSpec((BR, w), lambda i: (i, 0))


def _full_spec(shape):
    nd = len(shape)
    return pl.BlockSpec(shape, lambda i, _n=nd: (0,) * _n)


def _tc_a(s_on, c_on, s_vo, c_vo, x, wn_on, wr_on, wn_vo, wr_vo, b):
    grid = (N // BR,)
    return pl.pallas_call(
        _tc_a_body,
        grid=grid,
        in_specs=[_row_spec(D), _row_spec(D), _row_spec(D), _row_spec(D),
                  _row_spec(D), _full_spec((D, HID)), _full_spec((D, HID)),
                  _full_spec((D, HID)), _full_spec((D, HID)),
                  _full_spec((1, HID))],
        out_specs=[_row_spec(D), _row_spec(D)],
        out_shape=[jax.ShapeDtypeStruct((N, D), F32)] * 2,
    )(s_on, c_on, s_vo, c_vo, x, wn_on, wr_on, wn_vo, wr_vo, b)


def _tc_b(on0, on1, vo0, vo1, c_on, c_vo, h0, h1,
          wn_on, wr_on, wn_vo, wr_vo, b, lin_w, lin_b):
    grid = (N // BR,)
    return pl.pallas_call(
        _tc_b_body,
        grid=grid,
        in_specs=[_row_spec(D)] * 4 + [_row_spec(D)] * 2 + [_row_spec(D)] * 2
                 + [_full_spec((HID, HID))] * 4
                 + [_full_spec((1, HID)), _full_spec((HID, D)),
                    _full_spec((1, D))],
        out_specs=[_row_spec(D)],
        out_shape=[jax.ShapeDtypeStruct((N, D), F32)],
    )(on0, on1, vo0, vo1, c_on, c_vo, h0, h1,
      wn_on, wr_on, wn_vo, wr_vo, b, lin_w, lin_b)[0]


def _tc_c(p0, p1, c_on, h3, *w):
    grid = (N // BR,)
    wspecs = [_full_spec(a.shape) for a in w]
    return pl.pallas_call(
        _tc_c_body,
        grid=grid,
        in_specs=[_row_spec(D), _row_spec(D), _row_spec(D), _row_spec(D)]
                 + wspecs,
        out_specs=[_row_spec(OUT)],
        out_shape=[jax.ShapeDtypeStruct((N, OUT), F32)],
    )(p0, p1, c_on, h3, *w)[0]


def kernel(x_note, edge_index_onset, edge_index_voice, params):
    p = params
    src_on = edge_index_onset[0].astype(jnp.int32)
    dst_on = edge_index_onset[1].astype(jnp.int32)
    src_vo = edge_index_voice[0].astype(jnp.int32)
    dst_vo = edge_index_voice[1].astype(jnp.int32)

    src_cat = jnp.concatenate([src_on, src_vo])
    dst_cat = jnp.concatenate([dst_on, dst_vo])
    s_on, s_vo, c_on, c_vo = _SC_L0(x_note, src_cat, dst_cat)

    b0 = (p['l0_on_b'] + p['l0_vo_b']).reshape(1, HID)
    h0, h1 = _tc_a(s_on, c_on, s_vo, c_vo, x_note,
                   p['l0_on_Wn'], p['l0_on_Wr'], p['l0_vo_Wn'], p['l0_vo_Wr'],
                   b0)

    on0, on1, vo0, vo1 = _SC_L1(h0, h1, src_on, dst_on, src_vo, dst_vo)

    b1 = (p['l1_on_b'] + p['l1_vo_b']).reshape(1, HID)
    h3 = _tc_b(on0, on1, vo0, vo1, c_on, c_vo, h0, h1,
               p['l1_on_Wn'], p['l1_on_Wr'], p['l1_vo_Wn'], p['l1_vo_Wr'],
               b1, p['lin_W'], p['lin_b'].reshape(1, D))

    p0, p1 = _SC_POOL(h3, src_on, dst_on)

    r = lambda a: a.reshape(1, -1)
    out = _tc_c(p0, p1, c_on, h3,
                r(p['norm_g']), r(p['norm_b']),
                p['pm_W1'], r(p['pm_b1']), r(p['pm_ln_g']), r(p['pm_ln_b']),
                p['pm_W2'], r(p['pm_b2']),
                p['clf_W1'], r(p['clf_b1']),
                r(p['bn_g']), r(p['bn_b']), r(p['bn_rm']), r(p['bn_rv']),
                p['clf_W2'], r(p['clf_b2']))
    return out
